# baseline (speedup 1.0000x reference)
import functools

import jax
import jax.numpy as jnp
from jax import lax
from jax.experimental import pallas as pl
from jax.experimental.pallas import tpu as pltpu
from jax.experimental.pallas import tpu_sc as plsc


def _sc_gather(table, idx2, n_rows, feat):
    info = plsc.get_sparse_core_info()
    ncores = info.num_cores
    nw = ncores * info.num_subcores
    rows_per_w = n_rows // nw
    ch = 128
    n_ch = rows_per_w // ch
    mesh = plsc.VectorSubcoreMesh(core_axis_name="c", subcore_axis_name="s")

    @functools.partial(
        pl.kernel,
        mesh=mesh,
        out_type=jax.ShapeDtypeStruct((n_rows, feat), jnp.float32),
        scratch_types=[
            pltpu.VMEM((n_ch, ch), jnp.int32),
            pltpu.VMEM((ch, feat), jnp.float32),
            pltpu.VMEM((ch, feat), jnp.float32),
            pltpu.SemaphoreType.DMA,
            pltpu.SemaphoreType.DMA,
            pltpu.SemaphoreType.DMA,
            pltpu.SemaphoreType.DMA,
        ],
    )
    def gather_kernel(table_hbm, idx_hbm, out_hbm, idx_v, rows_a, rows_b,
                      gsem_a, gsem_b, osem_a, osem_b):
        wid = lax.axis_index("s") * ncores + lax.axis_index("c")
        base = wid * rows_per_w
        bufs = (rows_a, rows_b)
        gsems = (gsem_a, gsem_b)
        osems = (osem_a, osem_b)
        pltpu.sync_copy(idx_hbm.at[pl.ds(wid * n_ch, n_ch)], idx_v)
        gathers = [None] * n_ch
        stores = [None] * n_ch
        gathers[0] = pltpu.async_copy(
            table_hbm.at[idx_v.at[0]], bufs[0], gsems[0])
        for j in range(n_ch):
            b = j & 1
            gathers[j].wait()
            if j + 1 < n_ch:
                if j >= 1:
                    stores[j - 1].wait()
                gathers[j + 1] = pltpu.async_copy(
                    table_hbm.at[idx_v.at[j + 1]], bufs[1 - b], gsems[1 - b])
            stores[j] = pltpu.async_copy(
                bufs[b], out_hbm.at[pl.ds(base + j * ch, ch)], osems[b])
        stores[n_ch - 2].wait()
        stores[n_ch - 1].wait()

    return gather_kernel(table, idx2)


def _recurrence(emb, tok2, wih_t, whh_t, b_sum2, bhh_n2, wh1_t, b_h12,
                seq_len, nb, e, na, chunk):
    n_grid = seq_len // chunk

    def _sigm(x):
        return 0.5 * jnp.tanh(0.5 * x) + 0.5

    def body(emb_ref, tok_ref, wih_ref, whh_ref, bsum_ref, bhhn_ref,
             wh1_ref, bh1_ref, probs_ref, gi_ref, h_ref, dead_ref):
        i = pl.program_id(0)

        @pl.when(i == 0)
        def _init():
            h_ref[...] = jnp.zeros_like(h_ref)
            dead_ref[...] = jnp.zeros_like(dead_ref)

        gi_ref[...] = jnp.reshape(
            jnp.dot(emb_ref[...], wih_ref[...],
                    preferred_element_type=jnp.float32)
            + bsum_ref[...],
            (chunk, nb, 3 * e),
        )

        zt3 = (tok_ref[...] == 0).astype(jnp.float32)
        zero1 = jnp.zeros((1, nb, 1), jnp.float32)
        m3 = zt3
        sh = 1
        while sh < chunk:
            shifted = jnp.concatenate(
                [jnp.zeros((sh, nb, 1), jnp.float32), m3[:chunk - sh]], axis=0)
            m3 = jnp.maximum(m3, shifted)
            sh *= 2
        exc3 = jnp.concatenate([zero1, m3[:chunk - 1]], axis=0)
        dead3 = jnp.maximum(exc3, dead_ref[...])
        gi_ref[:, :, e:2 * e] = gi_ref[:, :, e:2 * e] + dead3 * 1e9
        dead_ref[...] = jnp.maximum(dead_ref[...], m3[chunk - 1:chunk])

        whh_rz = whh_ref[:, :2 * e]
        whh_n = whh_ref[:, 2 * e:]
        bhhn = bhhn_ref[...]

        def step(t, h):
            g = gi_ref[t]
            gh_rz = jnp.dot(h, whh_rz, preferred_element_type=jnp.float32)
            gh_n = jnp.dot(h, whh_n, preferred_element_type=jnp.float32)
            r = _sigm(g[:, :e] + gh_rz[:, :e])
            th = jnp.tanh(0.5 * (g[:, e:2 * e] + gh_rz[:, e:]))
            z = 0.5 * th + 0.5
            zc = 0.5 - 0.5 * th
            n = jnp.tanh(g[:, 2 * e:] + r * (gh_n + bhhn))
            return zc * n + z * h

        h = lax.fori_loop(0, chunk, step, h_ref[...], unroll=8)
        h_ref[...] = h

        @pl.when(i == n_grid - 1)
        def _final():
            logits = (jnp.dot(h, wh1_ref[...],
                              preferred_element_type=jnp.float32)
                      + bh1_ref[...])
            m = jnp.max(logits, axis=-1, keepdims=True)
            ex = jnp.exp(logits - m)
            probs_ref[...] = ex / jnp.sum(ex, axis=-1, keepdims=True)

    return pl.pallas_call(
        body,
        grid=(n_grid,),
        in_specs=[
            pl.BlockSpec((chunk * nb, e), lambda i: (i, 0)),
            pl.BlockSpec((chunk, nb, 1), lambda i: (i, 0, 0)),
            pl.BlockSpec((e, 3 * e), lambda i: (0, 0)),
            pl.BlockSpec((e, 3 * e), lambda i: (0, 0)),
            pl.BlockSpec((1, 3 * e), lambda i: (0, 0)),
            pl.BlockSpec((1, e), lambda i: (0, 0)),
            pl.BlockSpec((e, na), lambda i: (0, 0)),
            pl.BlockSpec((1, na), lambda i: (0, 0)),
        ],
        out_specs=pl.BlockSpec((nb, na), lambda i: (0, 0)),
        out_shape=jax.ShapeDtypeStruct((nb, na), jnp.float32),
        scratch_shapes=[
            pltpu.VMEM((chunk, nb, 3 * e), jnp.float32),
            pltpu.VMEM((nb, e), jnp.float32),
            pltpu.VMEM((1, nb, 1), jnp.float32),
        ],
        compiler_params=pltpu.CompilerParams(
            dimension_semantics=("arbitrary",),
        ),
    )(emb, tok2, wih_t, whh_t, b_sum2, bhh_n2, wh1_t, b_h12)


def kernel(utterance, global_idxes, d2e, W_ih, W_hh, b_ih, b_hh, W_h1, b_h1):
    nb, seq_len = utterance.shape
    e = W_hh.shape[1]
    na = W_h1.shape[0]

    toks_tm = utterance.T
    idx2 = toks_tm.reshape(-1, 128)
    emb = _sc_gather(d2e, idx2, nb * seq_len, e)


    b_sum = b_ih + jnp.concatenate([b_hh[:2 * e], jnp.zeros((e,), b_hh.dtype)])
    probs = _recurrence(
        emb, toks_tm.reshape(seq_len, nb, 1), W_ih.T, W_hh.T,
        b_sum.reshape(1, -1), b_hh[2 * e:].reshape(1, -1),
        W_h1.T, b_h1.reshape(1, -1),
        seq_len, nb, e, na, chunk=512,
    )

    skey = jax.random.key(1234)
    actions = jax.random.categorical(skey, jnp.log(probs + 1e-12), axis=-1)
    log_probs = jnp.log(
        jnp.take_along_axis(probs, actions[:, None], axis=1)[:, 0] + 1e-12)
    return actions, log_probs, probs


# speedup vs baseline: 1.0167x; 1.0167x over previous
import functools

import jax
import jax.numpy as jnp
from jax import lax
from jax.experimental import pallas as pl
from jax.experimental.pallas import tpu as pltpu
from jax.experimental.pallas import tpu_sc as plsc


def _sc_gather(table, idx2, n_rows, feat):
    info = plsc.get_sparse_core_info()
    ncores = info.num_cores
    nw = ncores * info.num_subcores
    rows_per_w = n_rows // nw
    ch = 128
    n_ch = rows_per_w // ch
    mesh = plsc.VectorSubcoreMesh(core_axis_name="c", subcore_axis_name="s")

    @functools.partial(
        pl.kernel,
        mesh=mesh,
        out_type=jax.ShapeDtypeStruct((n_rows, feat), jnp.float32),
        scratch_types=[
            pltpu.VMEM((n_ch, ch), jnp.int32),
            pltpu.VMEM((ch, feat), jnp.float32),
            pltpu.VMEM((ch, feat), jnp.float32),
            pltpu.SemaphoreType.DMA,
            pltpu.SemaphoreType.DMA,
            pltpu.SemaphoreType.DMA,
            pltpu.SemaphoreType.DMA,
        ],
    )
    def gather_kernel(table_hbm, idx_hbm, out_hbm, idx_v, rows_a, rows_b,
                      gsem_a, gsem_b, osem_a, osem_b):
        wid = lax.axis_index("s") * ncores + lax.axis_index("c")
        base = wid * rows_per_w
        bufs = (rows_a, rows_b)
        gsems = (gsem_a, gsem_b)
        osems = (osem_a, osem_b)
        pltpu.sync_copy(idx_hbm.at[pl.ds(wid * n_ch, n_ch)], idx_v)
        gathers = [None] * n_ch
        stores = [None] * n_ch
        gathers[0] = pltpu.async_copy(
            table_hbm.at[idx_v.at[0]], bufs[0], gsems[0])
        for j in range(n_ch):
            b = j & 1
            gathers[j].wait()
            if j + 1 < n_ch:
                if j >= 1:
                    stores[j - 1].wait()
                gathers[j + 1] = pltpu.async_copy(
                    table_hbm.at[idx_v.at[j + 1]], bufs[1 - b], gsems[1 - b])
            stores[j] = pltpu.async_copy(
                bufs[b], out_hbm.at[pl.ds(base + j * ch, ch)], osems[b])
        stores[n_ch - 2].wait()
        stores[n_ch - 1].wait()

    return gather_kernel(table, idx2)


def _recurrence(emb, tok2, wih_t, whh_t, b_sum2, bhh_n2, wh1_t, b_h12,
                seq_len, nb, e, na, chunk):
    n_grid = seq_len // chunk

    def _sigm(x):
        return 0.5 * jnp.tanh(0.5 * x) + 0.5

    def body(emb_ref, tok_ref, wih_ref, whh_ref, bsum_ref, bhhn_ref,
             wh1_ref, bh1_ref, probs_ref, gi_ref, h_ref, dead_ref):
        i = pl.program_id(0)

        @pl.when(i == 0)
        def _init():
            h_ref[...] = jnp.zeros_like(h_ref)
            dead_ref[...] = jnp.zeros_like(dead_ref)

        gi_ref[...] = jnp.reshape(
            jnp.dot(emb_ref[...], wih_ref[...],
                    preferred_element_type=jnp.float32)
            + bsum_ref[...],
            (chunk, nb, 3 * e),
        )

        zt3 = (tok_ref[...] == 0).astype(jnp.float32)
        need = (jnp.max(zt3) > 0.0) | (jnp.max(dead_ref[...]) > 0.0)

        @pl.when(need)
        def _sieve():
            zero1 = jnp.zeros((1, nb, 1), jnp.float32)
            m3 = zt3
            sh = 1
            while sh < chunk:
                shifted = jnp.concatenate(
                    [jnp.zeros((sh, nb, 1), jnp.float32), m3[:chunk - sh]],
                    axis=0)
                m3 = jnp.maximum(m3, shifted)
                sh *= 2
            exc3 = jnp.concatenate([zero1, m3[:chunk - 1]], axis=0)
            dead3 = jnp.maximum(exc3, dead_ref[...])
            gi_ref[:, :, e:2 * e] = gi_ref[:, :, e:2 * e] + dead3 * 1e9
            dead_ref[...] = jnp.maximum(dead_ref[...], m3[chunk - 1:chunk])

        whh_rz = whh_ref[:, :2 * e]
        whh_n = whh_ref[:, 2 * e:]
        bhhn = bhhn_ref[...]

        def step(t, h):
            g = gi_ref[t]
            gh_rz = jnp.dot(h, whh_rz, preferred_element_type=jnp.float32)
            gh_n = jnp.dot(h, whh_n, preferred_element_type=jnp.float32)
            r = _sigm(g[:, :e] + gh_rz[:, :e])
            th = jnp.tanh(0.5 * (g[:, e:2 * e] + gh_rz[:, e:]))
            z = 0.5 * th + 0.5
            zc = 0.5 - 0.5 * th
            n = jnp.tanh(g[:, 2 * e:] + r * (gh_n + bhhn))
            return zc * n + z * h

        h = lax.fori_loop(0, chunk, step, h_ref[...], unroll=8)
        h_ref[...] = h

        @pl.when(i == n_grid - 1)
        def _final():
            logits = (jnp.dot(h, wh1_ref[...],
                              preferred_element_type=jnp.float32)
                      + bh1_ref[...])
            m = jnp.max(logits, axis=-1, keepdims=True)
            ex = jnp.exp(logits - m)
            probs_ref[...] = ex / jnp.sum(ex, axis=-1, keepdims=True)

    return pl.pallas_call(
        body,
        grid=(n_grid,),
        in_specs=[
            pl.BlockSpec((chunk * nb, e), lambda i: (i, 0)),
            pl.BlockSpec((chunk, nb, 1), lambda i: (i, 0, 0)),
            pl.BlockSpec((e, 3 * e), lambda i: (0, 0)),
            pl.BlockSpec((e, 3 * e), lambda i: (0, 0)),
            pl.BlockSpec((1, 3 * e), lambda i: (0, 0)),
            pl.BlockSpec((1, e), lambda i: (0, 0)),
            pl.BlockSpec((e, na), lambda i: (0, 0)),
            pl.BlockSpec((1, na), lambda i: (0, 0)),
        ],
        out_specs=pl.BlockSpec((nb, na), lambda i: (0, 0)),
        out_shape=jax.ShapeDtypeStruct((nb, na), jnp.float32),
        scratch_shapes=[
            pltpu.VMEM((chunk, nb, 3 * e), jnp.float32),
            pltpu.VMEM((nb, e), jnp.float32),
            pltpu.VMEM((1, nb, 1), jnp.float32),
        ],
        compiler_params=pltpu.CompilerParams(
            dimension_semantics=("arbitrary",),
        ),
    )(emb, tok2, wih_t, whh_t, b_sum2, bhh_n2, wh1_t, b_h12)


def kernel(utterance, global_idxes, d2e, W_ih, W_hh, b_ih, b_hh, W_h1, b_h1):
    nb, seq_len = utterance.shape
    e = W_hh.shape[1]
    na = W_h1.shape[0]

    toks_tm = utterance.T
    idx2 = toks_tm.reshape(-1, 128)
    emb = _sc_gather(d2e, idx2, nb * seq_len, e)


    b_sum = b_ih + jnp.concatenate([b_hh[:2 * e], jnp.zeros((e,), b_hh.dtype)])
    probs = _recurrence(
        emb, toks_tm.reshape(seq_len, nb, 1), W_ih.T, W_hh.T,
        b_sum.reshape(1, -1), b_hh[2 * e:].reshape(1, -1),
        W_h1.T, b_h1.reshape(1, -1),
        seq_len, nb, e, na, chunk=512,
    )

    skey = jax.random.key(1234)
    actions = jax.random.categorical(skey, jnp.log(probs + 1e-12), axis=-1)
    log_probs = jnp.log(
        jnp.take_along_axis(probs, actions[:, None], axis=1)[:, 0] + 1e-12)
    return actions, log_probs, probs
